# Initial kernel scaffold; baseline (speedup 1.0000x reference)
import jax
import jax.numpy as jnp
from jax.experimental import pallas as pl


def kernel(x, A, Wg0_o, Wt0_o, Wg1_o, Wt1_o, Wg2_o, Wt2_o, Wf_o, Wg0_m, Wt0_m, Wg1_m, Wt1_m, Wg2_m, Wt2_m, Wf_m):
    raise NotImplementedError("write your pallas kernel here")


# speedup vs baseline: 27.1159x; 27.1159x over previous
import jax
import jax.numpy as jnp
from jax.experimental import pallas as pl

N_, C_, T_, V_, M_ = 4, 3, 300, 25, 2
K_ = 3
TK_ = 9
NCLS_ = 60
OFF_ = TK_ // 2


def _spatial(h3, At, Wg):
    v, tp, cin = h3.shape
    cout = Wg.shape[-1]
    h2 = h3.reshape(v * tp, cin)
    y = None
    for k in range(K_):
        p = jnp.dot(h2, Wg[k], preferred_element_type=jnp.float32)
        pv = (p.reshape(v, tp, cout) + 0.0).reshape(v, tp * cout)
        g = jnp.dot(At[k], pv, preferred_element_type=jnp.float32)
        y = g if y is None else y + g
    return y.reshape(v, tp, cout)


def _tconv_relu(y3, Wtp, stride, wacc, tvalid):
    v, tp, c = y3.shape
    acc = jnp.zeros((v * wacc, c), jnp.float32)
    if stride == 1:
        for tau in range(TK_):
            sl = y3[:, tau:tau + wacc, :].reshape(v * wacc, c)
            acc = acc + jnp.dot(sl, Wtp[tau], preferred_element_type=jnp.float32)
    else:
        ypp = y3.reshape(v, tp // 2, 2 * c)
        for tau in range(TK_):
            j = tau // 2
            lo = 0 if tau % 2 == 0 else c
            sl = ypp[:, j:j + wacc, lo:lo + c].reshape(v * wacc, c)
            acc = acc + jnp.dot(sl, Wtp[tau], preferred_element_type=jnp.float32)
    acc3 = jnp.maximum(acc, 0.0).reshape(v, wacc, c)
    tidx = jax.lax.broadcasted_iota(jnp.int32, (v, wacc, c), 1)
    return jnp.where(tidx < tvalid, acc3, 0.0)


def _repack(m3, tp_out):
    v, wacc, c = m3.shape
    zl = jnp.zeros((v, OFF_, c), jnp.float32)
    zr = jnp.zeros((v, tp_out - OFF_ - wacc, c), jnp.float32)
    return jnp.concatenate([zl, m3, zr], axis=1)


def _layer0_body(x_ref, At_ref, Wg_ref, Wtp_ref, out_ref):
    s = pl.program_id(0)
    x3 = x_ref[0]
    core = (x3[:, 5:303, :] - 0.5 * x3[:, 6:304, :] - 0.5 * x3[:, 4:302, :])
    zl = jnp.zeros((V_, 5, C_), jnp.float32)
    zr = jnp.zeros((V_, 9, C_), jnp.float32)
    mo = jnp.concatenate([zl, core, zr], axis=1)
    h = jnp.where(s == 0, x3, mo)
    y = _spatial(h, At_ref[...], Wg_ref[0])
    m = _tconv_relu(y, Wtp_ref[0], 1, 304, T_)
    out_ref[0, 0] = _repack(m, 312)


def _layer1_body(h_ref, At_ref, Wg_ref, Wtp_ref, out_ref):
    y = _spatial(h_ref[0, 0], At_ref[...], Wg_ref[0])
    m = _tconv_relu(y, Wtp_ref[0], 2, 152, T_ // 2)
    out_ref[0, 0] = _repack(m, 168)


def _layer2_body(h_ref, At_ref, Wg_ref, Wtp_ref, Wf_ref, out_ref):
    y = _spatial(h_ref[0, 0], At_ref[...], Wg_ref[0])
    m = _tconv_relu(y, Wtp_ref[0], 2, 80, T_ // 4)
    feat = jnp.sum(m.reshape(V_ * 80, 256), axis=0, keepdims=True)
    feat = feat * (1.0 / (V_ * (T_ // 4)))
    out_ref[0, 0] = jnp.dot(feat, Wf_ref[0], preferred_element_type=jnp.float32)


def kernel(x, A, Wg0_o, Wt0_o, Wg1_o, Wt1_o, Wg2_o, Wt2_o, Wf_o,
           Wg0_m, Wt0_m, Wg1_m, Wt1_m, Wg2_m, Wt2_m, Wf_m):
    if x.ndim == 4:
        x = x[..., None]
    nm = N_ * M_
    x8 = jnp.transpose(x, (0, 4, 3, 2, 1)).reshape(nm, V_, T_, C_)
    xc = jnp.pad(x8, ((0, 0), (0, 0), (OFF_, 8), (0, 0)))
    At = jnp.swapaxes(A, 1, 2)

    def tw(Wt):
        return jnp.transpose(Wt[:, :, :, 0], (2, 1, 0))

    Wg0 = jnp.stack([Wg0_o, Wg0_m])
    Wg1 = jnp.stack([Wg1_o, Wg1_m])
    Wg2 = jnp.stack([Wg2_o, Wg2_m])
    Wt0 = jnp.stack([tw(Wt0_o), tw(Wt0_m)])
    Wt1 = jnp.stack([tw(Wt1_o), tw(Wt1_m)])
    Wt2 = jnp.stack([tw(Wt2_o), tw(Wt2_m)])
    Wf = jnp.stack([Wf_o, Wf_m])

    grid = (2, nm)
    at_spec = pl.BlockSpec((K_, V_, V_), lambda s, i: (0, 0, 0))

    def wspec(shape):
        return pl.BlockSpec((1,) + shape, lambda s, i: (s,) + (0,) * len(shape))

    def hspec(tp, c):
        return pl.BlockSpec((1, 1, V_, tp, c), lambda s, i: (s, i, 0, 0, 0))

    h1 = pl.pallas_call(
        _layer0_body,
        grid=grid,
        in_specs=[
            pl.BlockSpec((1, V_, 312, C_), lambda s, i: (i, 0, 0, 0)),
            at_spec,
            wspec((K_, C_, 64)),
            wspec((TK_, 64, 64)),
        ],
        out_specs=hspec(312, 64),
        out_shape=jax.ShapeDtypeStruct((2, nm, V_, 312, 64), jnp.float32),
    )(xc, At, Wg0, Wt0)

    h2 = pl.pallas_call(
        _layer1_body,
        grid=grid,
        in_specs=[hspec(312, 64), at_spec, wspec((K_, 64, 128)),
                  wspec((TK_, 128, 128))],
        out_specs=hspec(168, 128),
        out_shape=jax.ShapeDtypeStruct((2, nm, V_, 168, 128), jnp.float32),
    )(h1, At, Wg1, Wt1)

    logits = pl.pallas_call(
        _layer2_body,
        grid=grid,
        in_specs=[hspec(168, 128), at_spec, wspec((K_, 128, 256)),
                  wspec((TK_, 256, 256)), wspec((256, NCLS_))],
        out_specs=pl.BlockSpec((1, 1, 1, NCLS_), lambda s, i: (s, i, 0, 0)),
        out_shape=jax.ShapeDtypeStruct((2, nm, 1, NCLS_), jnp.float32),
    )(h2, At, Wg2, Wt2, Wf)

    per_sample = logits[0, :, 0, :] + logits[1, :, 0, :]
    return per_sample.reshape(N_, M_, NCLS_).mean(axis=1)


# speedup vs baseline: 32.8865x; 1.2128x over previous
import jax
import jax.numpy as jnp
from jax.experimental import pallas as pl

N_, C_, T_, V_, M_ = 4, 3, 300, 25, 2
K_ = 3
TK_ = 9
NCLS_ = 60
OFF_ = TK_ // 2
BF = jnp.bfloat16


def _spatial(h3, At, Wg):
    v, tp, cin = h3.shape
    cout = Wg.shape[-1]
    h2 = h3.reshape(v * tp, cin)
    y = None
    for k in range(K_):
        p = jnp.dot(h2, Wg[k], preferred_element_type=jnp.float32)
        pv = (p.reshape(v, tp, cout) + 0.0).astype(BF).reshape(v, tp * cout)
        g = jnp.dot(At[k], pv, preferred_element_type=jnp.float32)
        y = g if y is None else y + g
    return y.reshape(v, tp, cout)


def _tconv_relu(y3, Wtp, stride, wacc, tvalid):
    v, tp, c = y3.shape
    yb = y3.astype(BF)
    acc = jnp.zeros((v * wacc, c), jnp.float32)
    if stride == 1:
        for tau in range(TK_):
            sl = yb[:, tau:tau + wacc, :].reshape(v * wacc, c)
            acc = acc + jnp.dot(sl, Wtp[tau], preferred_element_type=jnp.float32)
    else:
        ypp = yb.reshape(v, tp // 2, 2 * c)
        for tau in range(TK_):
            j = tau // 2
            lo = 0 if tau % 2 == 0 else c
            sl = ypp[:, j:j + wacc, lo:lo + c].reshape(v * wacc, c)
            acc = acc + jnp.dot(sl, Wtp[tau], preferred_element_type=jnp.float32)
    acc3 = jnp.maximum(acc, 0.0).reshape(v, wacc, c)
    tidx = jax.lax.broadcasted_iota(jnp.int32, (v, wacc, c), 1)
    return jnp.where(tidx < tvalid, acc3, 0.0)


def _repack(m3, tp_out):
    v, wacc, c = m3.shape
    zl = jnp.zeros((v, OFF_, c), BF)
    zr = jnp.zeros((v, tp_out - OFF_ - wacc, c), BF)
    return jnp.concatenate([zl, m3.astype(BF), zr], axis=1)


def _layer0_body(x_ref, At_ref, Wg_ref, Wtp_ref, out_ref):
    s = pl.program_id(0)
    x3 = x_ref[0]
    core = (x3.astype(jnp.float32)[:, 5:303, :]
            - 0.5 * x3.astype(jnp.float32)[:, 6:304, :]
            - 0.5 * x3.astype(jnp.float32)[:, 4:302, :]).astype(BF)
    zl = jnp.zeros((V_, 5, C_), BF)
    zr = jnp.zeros((V_, 9, C_), BF)
    mo = jnp.concatenate([zl, core, zr], axis=1)
    h = jnp.where(s == 0, x3, mo)
    y = _spatial(h, At_ref[...], Wg_ref[0])
    m = _tconv_relu(y, Wtp_ref[0], 1, 304, T_)
    out_ref[0, 0] = _repack(m, 312)


def _layer1_body(h_ref, At_ref, Wg_ref, Wtp_ref, out_ref):
    y = _spatial(h_ref[0, 0], At_ref[...], Wg_ref[0])
    m = _tconv_relu(y, Wtp_ref[0], 2, 152, T_ // 2)
    out_ref[0, 0] = _repack(m, 168)


def _layer2_body(h_ref, At_ref, Wg_ref, Wtp_ref, Wf_ref, out_ref):
    y = _spatial(h_ref[0, 0], At_ref[...], Wg_ref[0])
    m = _tconv_relu(y, Wtp_ref[0], 2, 80, T_ // 4)
    feat = jnp.sum(m.reshape(V_ * 80, 256), axis=0, keepdims=True)
    feat = feat * (1.0 / (V_ * (T_ // 4)))
    out_ref[0, 0] = jnp.dot(feat, Wf_ref[0], preferred_element_type=jnp.float32)


def kernel(x, A, Wg0_o, Wt0_o, Wg1_o, Wt1_o, Wg2_o, Wt2_o, Wf_o,
           Wg0_m, Wt0_m, Wg1_m, Wt1_m, Wg2_m, Wt2_m, Wf_m):
    if x.ndim == 4:
        x = x[..., None]
    nm = N_ * M_
    x8 = jnp.transpose(x, (0, 4, 3, 2, 1)).reshape(nm, V_, T_, C_)
    xc = jnp.pad(x8, ((0, 0), (0, 0), (OFF_, 8), (0, 0))).astype(BF)
    At = jnp.swapaxes(A, 1, 2).astype(BF)

    def tw(Wt):
        return jnp.transpose(Wt[:, :, :, 0], (2, 1, 0))

    Wg0 = jnp.stack([Wg0_o, Wg0_m]).astype(BF)
    Wg1 = jnp.stack([Wg1_o, Wg1_m]).astype(BF)
    Wg2 = jnp.stack([Wg2_o, Wg2_m]).astype(BF)
    Wt0 = jnp.stack([tw(Wt0_o), tw(Wt0_m)]).astype(BF)
    Wt1 = jnp.stack([tw(Wt1_o), tw(Wt1_m)]).astype(BF)
    Wt2 = jnp.stack([tw(Wt2_o), tw(Wt2_m)]).astype(BF)
    Wf = jnp.stack([Wf_o, Wf_m])

    grid = (2, nm)
    at_spec = pl.BlockSpec((K_, V_, V_), lambda s, i: (0, 0, 0))

    def wspec(shape):
        return pl.BlockSpec((1,) + shape, lambda s, i: (s,) + (0,) * len(shape))

    def hspec(tp, c):
        return pl.BlockSpec((1, 1, V_, tp, c), lambda s, i: (s, i, 0, 0, 0))

    h1 = pl.pallas_call(
        _layer0_body,
        grid=grid,
        in_specs=[
            pl.BlockSpec((1, V_, 312, C_), lambda s, i: (i, 0, 0, 0)),
            at_spec,
            wspec((K_, C_, 64)),
            wspec((TK_, 64, 64)),
        ],
        out_specs=hspec(312, 64),
        out_shape=jax.ShapeDtypeStruct((2, nm, V_, 312, 64), BF),
    )(xc, At, Wg0, Wt0)

    h2 = pl.pallas_call(
        _layer1_body,
        grid=grid,
        in_specs=[hspec(312, 64), at_spec, wspec((K_, 64, 128)),
                  wspec((TK_, 128, 128))],
        out_specs=hspec(168, 128),
        out_shape=jax.ShapeDtypeStruct((2, nm, V_, 168, 128), BF),
    )(h1, At, Wg1, Wt1)

    logits = pl.pallas_call(
        _layer2_body,
        grid=grid,
        in_specs=[hspec(168, 128), at_spec, wspec((K_, 128, 256)),
                  wspec((TK_, 256, 256)), wspec((256, NCLS_))],
        out_specs=pl.BlockSpec((1, 1, 1, NCLS_), lambda s, i: (s, i, 0, 0)),
        out_shape=jax.ShapeDtypeStruct((2, nm, 1, NCLS_), jnp.float32),
    )(h2, At, Wg2, Wt2, Wf)

    per_sample = logits[0, :, 0, :] + logits[1, :, 0, :]
    return per_sample.reshape(N_, M_, NCLS_).mean(axis=1)


# speedup vs baseline: 36.2812x; 1.1032x over previous
import jax
import jax.numpy as jnp
from jax.experimental import pallas as pl

N_, C_, T_, V_, M_ = 4, 3, 300, 25, 2
K_ = 3
TK_ = 9
NCLS_ = 60
OFF_ = TK_ // 2
BF = jnp.bfloat16


def _joint_chan(hm, As, Wg, tp, cin):
    g = jnp.dot(As, hm, preferred_element_type=jnp.float32)
    gb = g.astype(BF)
    g2 = (gb.reshape(K_ * V_, tp, cin) + 0.0).reshape(K_ * V_ * tp, cin)
    y = None
    for k in range(K_):
        blk = g2[k * V_ * tp:(k + 1) * V_ * tp]
        t = jnp.dot(blk, Wg[k], preferred_element_type=jnp.float32)
        y = t if y is None else y + t
    return y


def _tconv_relu(y3, Wtp, stride, wacc, tvalid):
    v, tp, c = y3.shape
    yb = y3.astype(BF)
    acc = jnp.zeros((v * wacc, c), jnp.float32)
    if stride == 1:
        for tau in range(TK_):
            sl = yb[:, tau:tau + wacc, :].reshape(v * wacc, c)
            acc = acc + jnp.dot(sl, Wtp[tau], preferred_element_type=jnp.float32)
    else:
        ypp = yb.reshape(v, tp // 2, 2 * c)
        for tau in range(TK_):
            j = tau // 2
            lo = 0 if tau % 2 == 0 else c
            sl = ypp[:, j:j + wacc, lo:lo + c].reshape(v * wacc, c)
            acc = acc + jnp.dot(sl, Wtp[tau], preferred_element_type=jnp.float32)
    acc3 = jnp.maximum(acc, 0.0).reshape(v, wacc, c)
    tidx = jax.lax.broadcasted_iota(jnp.int32, (v, wacc, c), 1)
    return jnp.where(tidx < tvalid, acc3, 0.0)


def _repack(m3, tp_out):
    v, wacc, c = m3.shape
    zl = jnp.zeros((v, OFF_, c), BF)
    zr = jnp.zeros((v, tp_out - OFF_ - wacc, c), BF)
    return jnp.concatenate([zl, m3.astype(BF), zr], axis=1)


def _layer0_body(hm_ref, As_ref, Wg_ref, Wtp_ref, out_ref):
    y = _joint_chan(hm_ref[0, 0], As_ref[...], Wg_ref[0], 312, C_)
    m = _tconv_relu(y.reshape(V_, 312, 64), Wtp_ref[0], 1, 304, T_)
    out_ref[0, 0] = _repack(m, 312)


def _layer1_body(hm_ref, As_ref, Wg_ref, Wtp_ref, out_ref):
    y = _joint_chan(hm_ref[0, 0], As_ref[...], Wg_ref[0], 312, 64)
    m = _tconv_relu(y.reshape(V_, 312, 128), Wtp_ref[0], 2, 152, T_ // 2)
    out_ref[0, 0] = _repack(m, 168)


def _layer2_body(hm_ref, As_ref, Wg_ref, Wtp_ref, Wf_ref, out_ref):
    y = _joint_chan(hm_ref[0, 0], As_ref[...], Wg_ref[0], 168, 128)
    m = _tconv_relu(y.reshape(V_, 168, 256), Wtp_ref[0], 2, 80, T_ // 4)
    feat = jnp.sum(m.reshape(V_ * 80, 256), axis=0, keepdims=True)
    feat = feat * (1.0 / (V_ * (T_ // 4)))
    out_ref[0, 0] = jnp.dot(feat, Wf_ref[0], preferred_element_type=jnp.float32)


def kernel(x, A, Wg0_o, Wt0_o, Wg1_o, Wt1_o, Wg2_o, Wt2_o, Wf_o,
           Wg0_m, Wt0_m, Wg1_m, Wt1_m, Wg2_m, Wt2_m, Wf_m):
    if x.ndim == 4:
        x = x[..., None]
    nm = N_ * M_
    x8 = jnp.transpose(x, (0, 4, 3, 2, 1)).reshape(nm, V_, T_, C_)
    core = x8[:, :, 1:-1] - 0.5 * x8[:, :, 2:] - 0.5 * x8[:, :, :-2]
    z1 = jnp.zeros((nm, V_, 1, C_), jnp.float32)
    mo = jnp.concatenate([z1, core, z1], axis=2)
    xs = jnp.stack([x8, mo])
    xc = jnp.pad(xs, ((0, 0), (0, 0), (0, 0), (OFF_, 8), (0, 0))).astype(BF)
    xm = xc.reshape(2, nm, V_, 312 * C_)
    As = jnp.concatenate([A[0].T, A[1].T, A[2].T], axis=0).astype(BF)

    def tw(Wt):
        return jnp.transpose(Wt[:, :, :, 0], (2, 1, 0))

    Wg0 = jnp.stack([Wg0_o, Wg0_m]).astype(BF)
    Wg1 = jnp.stack([Wg1_o, Wg1_m]).astype(BF)
    Wg2 = jnp.stack([Wg2_o, Wg2_m]).astype(BF)
    Wt0 = jnp.stack([tw(Wt0_o), tw(Wt0_m)]).astype(BF)
    Wt1 = jnp.stack([tw(Wt1_o), tw(Wt1_m)]).astype(BF)
    Wt2 = jnp.stack([tw(Wt2_o), tw(Wt2_m)]).astype(BF)
    Wf = jnp.stack([Wf_o, Wf_m])

    grid = (2, nm)
    as_spec = pl.BlockSpec((K_ * V_, V_), lambda s, i: (0, 0))

    def wspec(shape):
        return pl.BlockSpec((1,) + shape, lambda s, i: (s,) + (0,) * len(shape))

    def hmspec(width):
        return pl.BlockSpec((1, 1, V_, width), lambda s, i: (s, i, 0, 0))

    def cspec(tp, c):
        return pl.BlockSpec((1, 1, V_, tp, c), lambda s, i: (s, i, 0, 0, 0))

    h1 = pl.pallas_call(
        _layer0_body,
        grid=grid,
        in_specs=[hmspec(312 * C_), as_spec, wspec((K_, C_, 64)),
                  wspec((TK_, 64, 64))],
        out_specs=cspec(312, 64),
        out_shape=jax.ShapeDtypeStruct((2, nm, V_, 312, 64), BF),
    )(xm, As, Wg0, Wt0)

    h1m = h1.reshape(2, nm, V_, 312 * 64)
    h2 = pl.pallas_call(
        _layer1_body,
        grid=grid,
        in_specs=[hmspec(312 * 64), as_spec, wspec((K_, 64, 128)),
                  wspec((TK_, 128, 128))],
        out_specs=cspec(168, 128),
        out_shape=jax.ShapeDtypeStruct((2, nm, V_, 168, 128), BF),
    )(h1m, As, Wg1, Wt1)

    h2m = h2.reshape(2, nm, V_, 168 * 128)
    logits = pl.pallas_call(
        _layer2_body,
        grid=grid,
        in_specs=[hmspec(168 * 128), as_spec, wspec((K_, 128, 256)),
                  wspec((TK_, 256, 256)), wspec((256, NCLS_))],
        out_specs=pl.BlockSpec((1, 1, 1, NCLS_), lambda s, i: (s, i, 0, 0)),
        out_shape=jax.ShapeDtypeStruct((2, nm, 1, NCLS_), jnp.float32),
    )(h2m, As, Wg2, Wt2, Wf)

    per_sample = logits[0, :, 0, :] + logits[1, :, 0, :]
    return per_sample.reshape(N_, M_, NCLS_).mean(axis=1)


# speedup vs baseline: 47.6284x; 1.3128x over previous
import jax
import jax.numpy as jnp
from jax.experimental import pallas as pl

N_, C_, T_, V_, M_ = 4, 3, 300, 25, 2
K_ = 3
TK_ = 9
NCLS_ = 60
OFF_ = TK_ // 2
BF = jnp.bfloat16


def _joint_chan(hm, As, Wg, tp, cin):
    g = jnp.dot(As, hm, preferred_element_type=jnp.float32)
    gb = g.astype(BF)
    g2 = (gb.reshape(K_ * V_, tp, cin) + 0.0).reshape(K_ * V_ * tp, cin)
    gw = jnp.concatenate(
        [g2[k * V_ * tp:(k + 1) * V_ * tp] for k in range(K_)], axis=1)
    return jnp.dot(gw, Wg, preferred_element_type=jnp.float32)


def _tconv_relu(y3, Wtp, stride, wacc, tvalid):
    v, tp, c = y3.shape
    yb = y3.astype(BF)
    acc = jnp.zeros((v * wacc, c), jnp.float32)
    if stride == 1:
        for tau in range(0, TK_ - 1, 2):
            a = yb[:, tau:tau + wacc, :].reshape(v * wacc, c)
            b = yb[:, tau + 1:tau + 1 + wacc, :].reshape(v * wacc, c)
            sl = jnp.concatenate([a, b], axis=1)
            w2 = jnp.concatenate([Wtp[tau], Wtp[tau + 1]], axis=0)
            acc = acc + jnp.dot(sl, w2, preferred_element_type=jnp.float32)
        sl = yb[:, TK_ - 1:TK_ - 1 + wacc, :].reshape(v * wacc, c)
        acc = acc + jnp.dot(sl, Wtp[TK_ - 1],
                            preferred_element_type=jnp.float32)
    else:
        ypp = yb.reshape(v, tp // 2, 2 * c)
        for tau in range(TK_):
            j = tau // 2
            lo = 0 if tau % 2 == 0 else c
            sl = ypp[:, j:j + wacc, lo:lo + c].reshape(v * wacc, c)
            acc = acc + jnp.dot(sl, Wtp[tau], preferred_element_type=jnp.float32)
    acc3 = jnp.maximum(acc, 0.0).reshape(v, wacc, c)
    tidx = jax.lax.broadcasted_iota(jnp.int32, (v, wacc, c), 1)
    return jnp.where(tidx < tvalid, acc3, 0.0)


def _repack(m3, tp_out):
    v, wacc, c = m3.shape
    zl = jnp.zeros((v, OFF_, c), BF)
    zr = jnp.zeros((v, tp_out - OFF_ - wacc, c), BF)
    return jnp.concatenate([zl, m3.astype(BF), zr], axis=1)


def _layer0_body(hm_ref, As_ref, Wg_ref, Wtp_ref, out_ref):
    y = _joint_chan(hm_ref[0, 0], As_ref[...], Wg_ref[0], 312, C_)
    m = _tconv_relu(y.reshape(V_, 312, 64), Wtp_ref[0], 1, 304, T_)
    out_ref[0, 0] = _repack(m, 312)


def _layer1_body(hm_ref, As_ref, Wg_ref, Wtp_ref, out_ref):
    y = _joint_chan(hm_ref[0, 0], As_ref[...], Wg_ref[0], 312, 64)
    m = _tconv_relu(y.reshape(V_, 312, 128), Wtp_ref[0], 2, 152, T_ // 2)
    out_ref[0, 0] = _repack(m, 168)


def _layer2_body(hm_ref, As_ref, Wg_ref, Wtp_ref, Wf_ref, out_ref):
    y = _joint_chan(hm_ref[0, 0], As_ref[...], Wg_ref[0], 168, 128)
    m = _tconv_relu(y.reshape(V_, 168, 256), Wtp_ref[0], 2, 80, T_ // 4)
    feat = jnp.sum(m.reshape(V_ * 80, 256), axis=0, keepdims=True)
    feat = feat * (1.0 / (V_ * (T_ // 4)))
    out_ref[0, 0] = jnp.dot(feat, Wf_ref[0], preferred_element_type=jnp.float32)


def kernel(x, A, Wg0_o, Wt0_o, Wg1_o, Wt1_o, Wg2_o, Wt2_o, Wf_o,
           Wg0_m, Wt0_m, Wg1_m, Wt1_m, Wg2_m, Wt2_m, Wf_m):
    if x.ndim == 4:
        x = x[..., None]
    nm = N_ * M_
    x8 = jnp.transpose(x, (0, 4, 3, 2, 1)).reshape(nm, V_, T_, C_)
    core = x8[:, :, 1:-1] - 0.5 * x8[:, :, 2:] - 0.5 * x8[:, :, :-2]
    z1 = jnp.zeros((nm, V_, 1, C_), jnp.float32)
    mo = jnp.concatenate([z1, core, z1], axis=2)
    xs = jnp.stack([x8, mo])
    xc = jnp.pad(xs, ((0, 0), (0, 0), (0, 0), (OFF_, 8), (0, 0))).astype(BF)
    xm = xc.reshape(2, nm, V_, 312 * C_)
    As = jnp.concatenate([A[0].T, A[1].T, A[2].T], axis=0).astype(BF)

    def tw(Wt):
        return jnp.transpose(Wt[:, :, :, 0], (2, 1, 0))

    Wg0 = jnp.stack([Wg0_o, Wg0_m]).astype(BF).reshape(2, K_ * C_, 64)
    Wg1 = jnp.stack([Wg1_o, Wg1_m]).astype(BF).reshape(2, K_ * 64, 128)
    Wg2 = jnp.stack([Wg2_o, Wg2_m]).astype(BF).reshape(2, K_ * 128, 256)
    Wt0 = jnp.stack([tw(Wt0_o), tw(Wt0_m)]).astype(BF)
    Wt1 = jnp.stack([tw(Wt1_o), tw(Wt1_m)]).astype(BF)
    Wt2 = jnp.stack([tw(Wt2_o), tw(Wt2_m)]).astype(BF)
    Wf = jnp.stack([Wf_o, Wf_m])

    grid = (2, nm)
    as_spec = pl.BlockSpec((K_ * V_, V_), lambda s, i: (0, 0))

    def wspec(shape):
        return pl.BlockSpec((1,) + shape, lambda s, i: (s,) + (0,) * len(shape))

    def hmspec(width):
        return pl.BlockSpec((1, 1, V_, width), lambda s, i: (s, i, 0, 0))

    def cspec(tp, c):
        return pl.BlockSpec((1, 1, V_, tp, c), lambda s, i: (s, i, 0, 0, 0))

    h1 = pl.pallas_call(
        _layer0_body,
        grid=grid,
        in_specs=[hmspec(312 * C_), as_spec, wspec((K_ * C_, 64)),
                  wspec((TK_, 64, 64))],
        out_specs=cspec(312, 64),
        out_shape=jax.ShapeDtypeStruct((2, nm, V_, 312, 64), BF),
    )(xm, As, Wg0, Wt0)

    h1m = h1.reshape(2, nm, V_, 312 * 64)
    h2 = pl.pallas_call(
        _layer1_body,
        grid=grid,
        in_specs=[hmspec(312 * 64), as_spec, wspec((K_ * 64, 128)),
                  wspec((TK_, 128, 128))],
        out_specs=cspec(168, 128),
        out_shape=jax.ShapeDtypeStruct((2, nm, V_, 168, 128), BF),
    )(h1m, As, Wg1, Wt1)

    h2m = h2.reshape(2, nm, V_, 168 * 128)
    logits = pl.pallas_call(
        _layer2_body,
        grid=grid,
        in_specs=[hmspec(168 * 128), as_spec, wspec((K_ * 128, 256)),
                  wspec((TK_, 256, 256)), wspec((256, NCLS_))],
        out_specs=pl.BlockSpec((1, 1, 1, NCLS_), lambda s, i: (s, i, 0, 0)),
        out_shape=jax.ShapeDtypeStruct((2, nm, 1, NCLS_), jnp.float32),
    )(h2m, As, Wg2, Wt2, Wf)

    per_sample = logits[0, :, 0, :] + logits[1, :, 0, :]
    return per_sample.reshape(N_, M_, NCLS_).mean(axis=1)
